# baseline (speedup 1.0000x reference)
import functools

import jax
import jax.numpy as jnp
from jax import lax
from jax.experimental import pallas as pl
from jax.experimental.pallas import tpu as pltpu
from jax.experimental.pallas import tpu_sc as plsc

_B, _Z, _Y, _X = 2, 16, 64, 64
_BZ = _B * _Z
_YX = _Y * _X
_NSITE = _BZ * _YX
_CMID = 32
_EPS = 1e-3
_NW = 32
_CHUNK = 128


_SUP = 512


def _sc_gather_rows(table, idx2d, scale, shift):
    P = idx2d.shape[0] * _CHUNK
    per_w = P // _NW
    n_sup = per_w // _SUP
    mesh = plsc.VectorSubcoreMesh(core_axis_name="c", subcore_axis_name="s")

    @functools.partial(
        pl.kernel, mesh=mesh,
        out_type=jax.ShapeDtypeStruct((P, 32), jnp.float32),
        scratch_types=[
            pltpu.VMEM((4, _CHUNK), jnp.int32),
            pltpu.VMEM((_SUP, 32), jnp.float32),
            pltpu.VMEM((32,), jnp.float32),
            pltpu.VMEM((32,), jnp.float32),
            pltpu.SemaphoreType.DMA,
        ],
        compiler_params=pltpu.CompilerParams(use_tc_tiling_on_sc=False),
    )
    def k(table_h, idx_h, sc_h, sh_h, out_h, idx_v, rows_v, sc_v, sh_v, sem):
        wid = lax.axis_index("s") * 2 + lax.axis_index("c")
        base = wid * per_w
        pltpu.sync_copy(sc_h, sc_v)
        pltpu.sync_copy(sh_h, sh_v)
        sa = sc_v[pl.ds(0, 16)]
        sb = sc_v[pl.ds(16, 16)]
        ha = sh_v[pl.ds(0, 16)]
        hb = sh_v[pl.ds(16, 16)]

        def sup(t, carry):
            b = base + t * _SUP
            pltpu.sync_copy(idx_h.at[pl.ds(b // _CHUNK, 4)], idx_v)
            hs = [pltpu.async_copy(
                table_h.at[idx_v.at[kk]],
                rows_v.at[pl.ds(kk * _CHUNK, _CHUNK)], sem)
                for kk in range(4)]
            for h in hs:
                h.wait()

            def row(r, c2):
                a = rows_v[r, pl.ds(0, 16)]
                rows_v[r, pl.ds(0, 16)] = jnp.maximum(a * sa + ha, 0.0)
                bv = rows_v[r, pl.ds(16, 16)]
                rows_v[r, pl.ds(16, 16)] = jnp.maximum(bv * sb + hb, 0.0)
                return c2

            lax.fori_loop(0, _SUP, row, 0)
            pltpu.sync_copy(rows_v, out_h.at[pl.ds(b, _SUP)])
            return carry

        lax.fori_loop(0, n_sup, sup, 0)

    return k(table, idx2d, scale, shift)


_NSL = 34
_NROW = _NSL * _YX
_HALF = 17 * _YX
_P0 = 32768


def _sc_scatter_dense(vf_pad, idx2c):
    per_tile = _HALF // 16
    n_sup = _P0 // (16 * _SUP)
    mesh = plsc.VectorSubcoreMesh(core_axis_name="c", subcore_axis_name="s")

    @functools.partial(
        pl.kernel, mesh=mesh,
        out_type=(jax.ShapeDtypeStruct((_NROW, 16), jnp.float32),
                  jax.ShapeDtypeStruct((_NROW, 16), jnp.float32)),
        scratch_types=[
            pltpu.VMEM((_SUP, 16), jnp.float32),
            pltpu.VMEM((_CHUNK, 16), jnp.float32),
            pltpu.VMEM((4, _CHUNK), jnp.int32),
            pltpu.VMEM((_SUP, 16), jnp.float32),
            pltpu.SemaphoreType.DMA,
        ],
        compiler_params=pltpu.CompilerParams(use_tc_tiling_on_sc=False),
    )
    def k(vf_h, idx_h, feat_h, mask_h, zb_v, ob_v, idx_v, rows_v, sem):
        c = lax.axis_index("c")
        s = lax.axis_index("s")

        def init(i, carry):
            zb_v[i, :] = jnp.zeros((16,), jnp.float32)
            return carry

        lax.fori_loop(0, _SUP, init, 0)

        def init1(i, carry):
            ob_v[i, :] = jnp.ones((16,), jnp.float32)
            return carry

        lax.fori_loop(0, _CHUNK, init1, 0)

        fb = c * _HALF + s * per_tile
        def fill(i, carry):
            r = fb + i * _SUP
            pltpu.sync_copy(zb_v, feat_h.at[pl.ds(r, _SUP)])
            pltpu.sync_copy(zb_v, mask_h.at[pl.ds(r, _SUP)])
            return carry

        lax.fori_loop(0, per_tile // _SUP, fill, 0)
        rrem = fb + (per_tile // _SUP) * _SUP
        pltpu.sync_copy(zb_v.at[pl.ds(0, per_tile % _SUP)],
                        feat_h.at[pl.ds(rrem, per_tile % _SUP)])
        pltpu.sync_copy(zb_v.at[pl.ds(0, per_tile % _SUP)],
                        mask_h.at[pl.ds(rrem, per_tile % _SUP)])
        plsc.subcore_barrier()

        sb = s * (_P0 // 16)
        def sup(t, carry):
            b = sb + t * _SUP
            pltpu.sync_copy(idx_h.at[c, pl.ds(b // _CHUNK, 4)], idx_v)
            pltpu.sync_copy(vf_h.at[pl.ds(b, _SUP)], rows_v)
            hs = []
            for kk in range(4):
                hs.append(pltpu.async_copy(
                    rows_v.at[pl.ds(kk * _CHUNK, _CHUNK)],
                    feat_h.at[idx_v.at[kk]], sem))
                hs.append(pltpu.async_copy(
                    ob_v, mask_h.at[idx_v.at[kk]], sem))
            for h in hs:
                h.wait()
            return carry

        lax.fori_loop(0, n_sup, sup, 0)

    return k(vf_pad, idx2c)


def _shift_lanes(x, o):
    c = x.shape[0]
    if o > 0:
        return jnp.concatenate(
            [x[:, o:], jnp.zeros((c, o), jnp.float32)], axis=1)
    if o < 0:
        return jnp.concatenate(
            [jnp.zeros((c, -o), jnp.float32), x[:, :o]], axis=1)
    return x


def _conv_body(kind, cin, *refs):
    scsh_ref = None
    if kind == "first":
        (x_in, m_in, w_ref, out_ref, maskout_ref, stats_ref,
         ring_x, ring_m) = refs
    elif kind == "dilate":
        (x_in, m_in, st_ref, g_ref, b_ref, w_ref,
         out_ref, maskout_ref, stats_ref, ring_x, ring_m) = refs
    elif kind == "last":
        (x_in, m_in, st_ref, g_ref, b_ref, gn_ref, bn_ref, w_ref,
         out_ref, stats_ref, scsh_ref, ring_x, ring_m) = refs
        maskout_ref = None
    else:
        (x_in, m_in, st_ref, g_ref, b_ref, w_ref,
         out_ref, stats_ref, ring_x, ring_m) = refs
        maskout_ref = None

    i = pl.program_id(0)
    lanes = lax.broadcasted_iota(jnp.int32, (1, _YX), 1) % _X
    xmask_m = (lanes != 0).astype(jnp.float32)
    xmask_p = (lanes != (_X - 1)).astype(jnp.float32)
    dilate = kind in ("first", "dilate")

    if kind == "first":
        feats = x_in[0].T
        m_raw = m_in[0].T[0:1]
    else:
        st = st_ref[...]
        mu = st[1:2] / st[0:1]
        var = st[2:3] / st[0:1] - mu * mu
        scl = g_ref[...] * lax.rsqrt(var + _EPS)
        shf = b_ref[...] - mu * scl
        m_raw = m_in[0]
        feats = jnp.maximum(
            x_in[0] * scl.T + shf.T, 0.0) * m_raw
    xstack = jnp.concatenate(
        [_shift_lanes(feats, -1) * xmask_m, feats,
         _shift_lanes(feats, 1) * xmask_p], axis=0)
    if dilate:
        mmax = jnp.maximum(
            jnp.maximum(_shift_lanes(m_raw, -1) * xmask_m, m_raw),
            _shift_lanes(m_raw, 1) * xmask_p)
        mstack = jnp.concatenate([m_raw, mmax], axis=0)
    else:
        mstack = m_raw
    mrows = mstack.shape[0]
    sl_new = jnp.remainder(i, 3)
    ring_x[pl.ds(sl_new, 1)] = xstack.reshape(1, 3 * cin, _YX)
    ring_m[pl.ds(sl_new, 1)] = mstack.reshape(1, mrows, _YX)

    @pl.when(i == 0)
    def _():
        ring_x[pl.ds(2, 1)] = jnp.zeros((1, 3 * cin, _YX), jnp.float32)
        ring_m[pl.ds(2, 1)] = jnp.zeros((1, mrows, _YX), jnp.float32)
        stats_ref[...] = jnp.zeros((3, _CMID), jnp.float32)

    @pl.when(i > 0)
    def _():
        zi = i - 1
        vzm = jnp.where((zi % _Z) != 0, 1.0, 0.0)
        vzp = jnp.where((zi % _Z) != (_Z - 1), 1.0, 0.0)
        s_m = jnp.remainder(i - 2, 3)
        s_0 = jnp.remainder(i - 1, 3)
        s_p = sl_new
        xcat = jnp.concatenate(
            [ring_x[pl.ds(s_m, 1)][0] * vzm,
             ring_x[pl.ds(s_0, 1)][0],
             ring_x[pl.ds(s_p, 1)][0] * vzp], axis=0)
        acc = None
        for dyi, dy in enumerate((-1, 0, 1)):
            k0 = dyi * 9 * cin
            if kind == "last":
                s_dy = lax.dot_general(
                    xcat, w_ref[:, k0:k0 + 9 * cin],
                    (((0,), (1,)), ((), ())),
                    preferred_element_type=jnp.float32)
                if dy:
                    c4 = jnp.zeros((_X, _CMID), jnp.float32)
                    s_dy = (jnp.concatenate([s_dy[_X:], c4], axis=0) if dy > 0
                            else jnp.concatenate([c4, s_dy[:-_X]], axis=0))
            else:
                s_dy = lax.dot_general(
                    w_ref[:, k0:k0 + 9 * cin], xcat,
                    (((1,), (0,)), ((), ())),
                    preferred_element_type=jnp.float32)
                s_dy = _shift_lanes(s_dy, _X * dy)
            acc = s_dy if acc is None else acc + s_dy

        if dilate:
            mdz = jnp.maximum(
                jnp.maximum(ring_m[pl.ds(s_m, 1)][0, 1:2] * vzm,
                            ring_m[pl.ds(s_0, 1)][0, 1:2]),
                ring_m[pl.ds(s_p, 1)][0, 1:2] * vzp)
            m_out = jnp.maximum(
                jnp.maximum(_shift_lanes(mdz, -_X), mdz),
                _shift_lanes(mdz, _X))
            maskout_ref[0] = m_out
        else:
            m_out = ring_m[pl.ds(s_0, 1)][0, 0:1]

        out_ref[0] = acc
        n = jnp.sum(m_out)
        if kind == "last":
            mcol = m_out.T
            s1 = jnp.sum(acc * mcol, axis=0)
            s2 = jnp.sum(acc * acc * mcol, axis=0)
        else:
            s1 = jnp.sum(acc * m_out, axis=1)
            s2 = jnp.sum(acc * acc * m_out, axis=1)
        stats_ref[...] += jnp.concatenate(
            [jnp.broadcast_to(n, (1, _CMID)),
             s1.reshape(1, _CMID), s2.reshape(1, _CMID)], axis=0)

        if kind == "last":
            @pl.when(i == _BZ)
            def _():
                st6 = stats_ref[...]
                mu6 = st6[1:2] / st6[0:1]
                var6 = st6[2:3] / st6[0:1] - mu6 * mu6
                sc6 = gn_ref[...] * lax.rsqrt(var6 + _EPS)
                scsh_ref[...] = jnp.concatenate(
                    [sc6, bn_ref[...] - mu6 * sc6], axis=0)


def _conv_layer(x, mask, scale, shift, wmat, kind, cin):
    def zin(i):
        return (jnp.clip(i, 0, _BZ - 1), 0, 0)

    def zin34(i):
        t = jnp.clip(i, 0, _BZ - 1)
        return (t + (t >= 16).astype(jnp.int32), 0, 0)

    def zout(i):
        return (jnp.clip(i - 1, 0, _BZ - 1), 0, 0)

    if kind == "first":
        xspec = pl.BlockSpec((1, _YX, 16), zin34)
        mspec = pl.BlockSpec((1, _YX, 16), zin34)
    else:
        xspec = pl.BlockSpec((1, _CMID, _YX), zin)
        mspec = pl.BlockSpec((1, 1, _YX), zin)
    full2 = lambda shape: pl.BlockSpec(shape, lambda i: (0, 0))

    in_specs = [xspec, mspec]
    inputs = [x, mask]
    if kind != "first":
        st_prev, g_prev, b_prev = scale
        in_specs += [full2((3, cin)), full2((1, cin)), full2((1, cin))]
        inputs += [st_prev, g_prev.reshape(1, cin), b_prev.reshape(1, cin)]
    if kind == "last":
        gn, bn = shift
        in_specs += [full2((1, _CMID)), full2((1, _CMID))]
        inputs += [gn.reshape(1, _CMID), bn.reshape(1, _CMID)]
    in_specs.append(full2(wmat.shape))
    inputs.append(wmat)
    mrows = 2 if kind in ("first", "dilate") else 1
    scratch_shapes = [
        pltpu.VMEM((3, 3 * cin, _YX), jnp.float32),
        pltpu.VMEM((3, mrows, _YX), jnp.float32),
    ]

    if kind == "last":
        out_shape = [jax.ShapeDtypeStruct((_BZ, _YX, _CMID), jnp.float32)]
        out_specs = [pl.BlockSpec((1, _YX, _CMID), zout)]
    else:
        out_shape = [jax.ShapeDtypeStruct((_BZ, _CMID, _YX), jnp.float32)]
        out_specs = [pl.BlockSpec((1, _CMID, _YX), zout)]
    if kind in ("first", "dilate"):
        out_shape.append(jax.ShapeDtypeStruct((_BZ, 1, _YX), jnp.float32))
        out_specs.append(pl.BlockSpec((1, 1, _YX), zout))
    out_shape.append(jax.ShapeDtypeStruct((3, _CMID), jnp.float32))
    out_specs.append(pl.BlockSpec((3, _CMID), lambda i: (0, 0)))
    if kind == "last":
        out_shape.append(jax.ShapeDtypeStruct((2, _CMID), jnp.float32))
        out_specs.append(pl.BlockSpec((2, _CMID), lambda i: (0, 0)))

    outs = pl.pallas_call(
        functools.partial(_conv_body, kind, cin),
        grid=(_BZ + 1,),
        in_specs=in_specs,
        out_specs=out_specs,
        out_shape=out_shape,
        scratch_shapes=scratch_shapes,
    )(*inputs)

    if kind in ("first", "dilate"):
        out, mask_out, stats = outs
        return out, mask_out, stats
    if kind == "last":
        out, stats, scsh = outs
        return out, scsh, stats
    out, stats = outs
    return out, mask, stats


def _flatten_coords(c):
    return ((c[:, 0] * _Z + c[:, 1]) * _Y + c[:, 2]) * _X + c[:, 3]


def kernel(voxel_features, voxel_indices, coords1, coords2, W1a, g1a, b1a,
           W1b, g1b, b1b, W1c, g1c, b1c, W2a, g2a, b2a, W2b, g2b, b2b,
           W2c, g2c, b2c):
    n0 = voxel_features.shape[0]
    n2 = coords2.shape[0]

    flat0 = _flatten_coords(voxel_indices)
    row = flat0 + _YX * (flat0 >= 16 * _YX).astype(jnp.int32)
    rowp = jnp.concatenate(
        [row, jnp.full((_P0 - n0,), -1, jnp.int32)])
    jm = jnp.arange(_P0, dtype=jnp.int32) % _YX
    idx2c = jnp.stack([
        jnp.where((rowp >= 0) & (rowp < _HALF), rowp, 16 * _YX + jm),
        jnp.where(rowp >= _HALF, rowp, 33 * _YX + jm),
    ]).reshape(2, _P0 // _CHUNK, _CHUNK)
    vf_pad = jnp.concatenate(
        [voxel_features, jnp.zeros((_P0 - n0, 16), jnp.float32)])

    feat_d, mask_d = _sc_scatter_dense(vf_pad, idx2c)
    x0 = feat_d.reshape(_NSL, _YX, 16)
    m0 = mask_d.reshape(_NSL, _YX, 16)

    def wm(W, cin):
        return W.reshape(3, 3, 3, cin, _CMID).transpose(
            1, 0, 2, 3, 4).reshape(27 * cin, _CMID).T

    o1, m1, s1 = _conv_layer(x0, m0, None, None, wm(W1a, 16), "first", 16)
    o2, _, s2 = _conv_layer(o1, m1, (s1, g1a, b1a), None,
                            wm(W1b, _CMID), "mid", _CMID)
    o3, _, s3 = _conv_layer(o2, m1, (s2, g1b, b1b), None,
                            wm(W1c, _CMID), "mid", _CMID)
    o4, m2, s4 = _conv_layer(o3, m1, (s3, g1c, b1c), None,
                             wm(W2a, _CMID), "dilate", _CMID)
    o5, _, s5 = _conv_layer(o4, m2, (s4, g2a, b2a), None,
                            wm(W2b, _CMID), "mid", _CMID)
    o6, scsh6, s6 = _conv_layer(o5, m2, (s5, g2b, b2b), (g2c, b2c),
                                wm(W2c, _CMID), "last", _CMID)

    flat2 = _flatten_coords(coords2)
    p2 = ((n2 + _NW * _SUP - 1) // (_NW * _SUP)) * (_NW * _SUP)
    idx2 = jnp.concatenate(
        [flat2, jnp.zeros((p2 - n2,), jnp.int32)]) if p2 != n2 else flat2
    rows6 = o6.reshape(_NSITE, _CMID)
    out = _sc_gather_rows(rows6, idx2.reshape(p2 // _CHUNK, _CHUNK),
                          scsh6[0], scsh6[1])
    return out[:n2]


# speedup vs baseline: 1.0391x; 1.0391x over previous
import functools

import jax
import jax.numpy as jnp
from jax import lax
from jax.experimental import pallas as pl
from jax.experimental.pallas import tpu as pltpu
from jax.experimental.pallas import tpu_sc as plsc

_B, _Z, _Y, _X = 2, 16, 64, 64
_BZ = _B * _Z
_YX = _Y * _X
_NSITE = _BZ * _YX
_CMID = 32
_EPS = 1e-3
_NW = 32
_CHUNK = 128


_SUP = 512


def _sc_gather_rows(table, idx2d, scale, shift):
    P = idx2d.shape[0] * _CHUNK
    per_w = P // _NW
    n_sup = per_w // _SUP
    mesh = plsc.VectorSubcoreMesh(core_axis_name="c", subcore_axis_name="s")

    @functools.partial(
        pl.kernel, mesh=mesh,
        out_type=jax.ShapeDtypeStruct((P, 32), jnp.float32),
        scratch_types=[
            pltpu.VMEM((4, _CHUNK), jnp.int32),
            pltpu.VMEM((_SUP, 32), jnp.float32),
            pltpu.VMEM((32,), jnp.float32),
            pltpu.VMEM((32,), jnp.float32),
            pltpu.SemaphoreType.DMA,
        ],
        compiler_params=pltpu.CompilerParams(use_tc_tiling_on_sc=False),
    )
    def k(table_h, idx_h, sc_h, sh_h, out_h, idx_v, rows_v, sc_v, sh_v, sem):
        wid = lax.axis_index("s") * 2 + lax.axis_index("c")
        base = wid * per_w
        pltpu.sync_copy(sc_h, sc_v)
        pltpu.sync_copy(sh_h, sh_v)
        sa = sc_v[pl.ds(0, 16)]
        sb = sc_v[pl.ds(16, 16)]
        ha = sh_v[pl.ds(0, 16)]
        hb = sh_v[pl.ds(16, 16)]

        def sup(t, carry):
            b = base + t * _SUP
            pltpu.sync_copy(idx_h.at[pl.ds(b // _CHUNK, 4)], idx_v)
            hs = [pltpu.async_copy(
                table_h.at[idx_v.at[kk]],
                rows_v.at[pl.ds(kk * _CHUNK, _CHUNK)], sem)
                for kk in range(4)]
            for h in hs:
                h.wait()

            def row(r, c2):
                a = rows_v[r, pl.ds(0, 16)]
                rows_v[r, pl.ds(0, 16)] = jnp.maximum(a * sa + ha, 0.0)
                bv = rows_v[r, pl.ds(16, 16)]
                rows_v[r, pl.ds(16, 16)] = jnp.maximum(bv * sb + hb, 0.0)
                return c2

            lax.fori_loop(0, _SUP, row, 0)
            pltpu.sync_copy(rows_v, out_h.at[pl.ds(b, _SUP)])
            return carry

        lax.fori_loop(0, n_sup, sup, 0)

    return k(table, idx2d, scale, shift)


_NSL = 34
_NROW = _NSL * _YX
_HALF = 17 * _YX
_P0 = 32768


def _sc_scatter_dense(vf_pad, idx2c):
    per_tile = _HALF // 16
    n_sup = _P0 // (16 * _SUP)
    mesh = plsc.VectorSubcoreMesh(core_axis_name="c", subcore_axis_name="s")

    @functools.partial(
        pl.kernel, mesh=mesh,
        out_type=(jax.ShapeDtypeStruct((_NROW, 16), jnp.float32),
                  jax.ShapeDtypeStruct((_NROW, 16), jnp.float32)),
        scratch_types=[
            pltpu.VMEM((_SUP, 16), jnp.float32),
            pltpu.VMEM((_CHUNK, 16), jnp.float32),
            pltpu.VMEM((4, _CHUNK), jnp.int32),
            pltpu.VMEM((_SUP, 16), jnp.float32),
            pltpu.SemaphoreType.DMA,
        ],
        compiler_params=pltpu.CompilerParams(use_tc_tiling_on_sc=False),
    )
    def k(vf_h, idx_h, feat_h, mask_h, zb_v, ob_v, idx_v, rows_v, sem):
        c = lax.axis_index("c")
        s = lax.axis_index("s")

        def init(i, carry):
            zb_v[i, :] = jnp.zeros((16,), jnp.float32)
            return carry

        lax.fori_loop(0, _SUP, init, 0)

        def init1(i, carry):
            ob_v[i, :] = jnp.ones((16,), jnp.float32)
            return carry

        lax.fori_loop(0, _CHUNK, init1, 0)

        fb = c * _HALF + s * per_tile
        def fill(i, carry):
            r = fb + i * _SUP
            pltpu.sync_copy(zb_v, feat_h.at[pl.ds(r, _SUP)])
            pltpu.sync_copy(zb_v, mask_h.at[pl.ds(r, _SUP)])
            return carry

        lax.fori_loop(0, per_tile // _SUP, fill, 0)
        rrem = fb + (per_tile // _SUP) * _SUP
        pltpu.sync_copy(zb_v.at[pl.ds(0, per_tile % _SUP)],
                        feat_h.at[pl.ds(rrem, per_tile % _SUP)])
        pltpu.sync_copy(zb_v.at[pl.ds(0, per_tile % _SUP)],
                        mask_h.at[pl.ds(rrem, per_tile % _SUP)])
        plsc.subcore_barrier()

        sb = s * (_P0 // 16)
        def sup(t, carry):
            b = sb + t * _SUP
            pltpu.sync_copy(idx_h.at[c, pl.ds(b // _CHUNK, 4)], idx_v)
            pltpu.sync_copy(vf_h.at[pl.ds(b, _SUP)], rows_v)
            hs = []
            for kk in range(4):
                hs.append(pltpu.async_copy(
                    rows_v.at[pl.ds(kk * _CHUNK, _CHUNK)],
                    feat_h.at[idx_v.at[kk]], sem))
                hs.append(pltpu.async_copy(
                    ob_v, mask_h.at[idx_v.at[kk]], sem))
            for h in hs:
                h.wait()
            return carry

        lax.fori_loop(0, n_sup, sup, 0)

    return k(vf_pad, idx2c)


def _shift_lanes(x, o):
    c = x.shape[0]
    if o > 0:
        return jnp.concatenate(
            [x[:, o:], jnp.zeros((c, o), jnp.float32)], axis=1)
    if o < 0:
        return jnp.concatenate(
            [jnp.zeros((c, -o), jnp.float32), x[:, :o]], axis=1)
    return x


def _conv_body(kind, cin, *refs):
    scsh_ref = None
    if kind == "first":
        (x_in, m_in, w_ref, out_ref, maskout_ref, stats_ref,
         ring_x, ring_m) = refs
    elif kind == "dilate":
        (x_in, m_in, st_ref, g_ref, b_ref, w_ref,
         out_ref, maskout_ref, stats_ref, ring_x, ring_m) = refs
    elif kind == "last":
        (x_in, m_in, st_ref, g_ref, b_ref, gn_ref, bn_ref, w_ref,
         out_ref, stats_ref, scsh_ref, ring_x, ring_m) = refs
        maskout_ref = None
    else:
        (x_in, m_in, st_ref, g_ref, b_ref, w_ref,
         out_ref, stats_ref, ring_x, ring_m) = refs
        maskout_ref = None

    i = pl.program_id(0)
    lanes = lax.broadcasted_iota(jnp.int32, (1, _YX), 1) % _X
    xmask_m = (lanes != 0).astype(jnp.float32)
    xmask_p = (lanes != (_X - 1)).astype(jnp.float32)
    dilate = kind in ("first", "dilate")

    if kind == "first":
        feats = x_in[0].T
        m_raw = m_in[0].T[0:1]
    else:
        st = st_ref[...]
        mu = st[1:2] / st[0:1]
        var = st[2:3] / st[0:1] - mu * mu
        scl = g_ref[...] * lax.rsqrt(var + _EPS)
        shf = b_ref[...] - mu * scl
        m_raw = m_in[0]
        feats = jnp.maximum(
            x_in[0] * scl.T + shf.T, 0.0) * m_raw
    xstack = jnp.concatenate(
        [_shift_lanes(feats, -1) * xmask_m, feats,
         _shift_lanes(feats, 1) * xmask_p], axis=0)
    if dilate:
        mmax = jnp.maximum(
            jnp.maximum(_shift_lanes(m_raw, -1) * xmask_m, m_raw),
            _shift_lanes(m_raw, 1) * xmask_p)
        mstack = jnp.concatenate([m_raw, mmax], axis=0)
    else:
        mstack = m_raw
    mrows = mstack.shape[0]
    sl_new = jnp.remainder(i, 3)
    ring_x[pl.ds(sl_new, 1)] = xstack.reshape(1, 3 * cin, _YX)
    ring_m[pl.ds(sl_new, 1)] = mstack.reshape(1, mrows, _YX)

    @pl.when(i == 0)
    def _():
        ring_x[pl.ds(2, 1)] = jnp.zeros((1, 3 * cin, _YX), jnp.float32)
        ring_m[pl.ds(2, 1)] = jnp.zeros((1, mrows, _YX), jnp.float32)
        stats_ref[...] = jnp.zeros((3, _CMID), jnp.float32)

    @pl.when(i > 0)
    def _():
        zi = i - 1
        vzm = jnp.where((zi % _Z) != 0, 1.0, 0.0)
        vzp = jnp.where((zi % _Z) != (_Z - 1), 1.0, 0.0)
        s_m = jnp.remainder(i - 2, 3)
        s_0 = jnp.remainder(i - 1, 3)
        s_p = sl_new
        xcat = jnp.concatenate(
            [ring_x[pl.ds(s_m, 1)][0] * vzm,
             ring_x[pl.ds(s_0, 1)][0],
             ring_x[pl.ds(s_p, 1)][0] * vzp], axis=0)
        acc = None
        for dyi, dy in enumerate((-1, 0, 1)):
            k0 = dyi * 9 * cin
            s_dy = lax.dot_general(
                w_ref[:, k0:k0 + 9 * cin], xcat, (((1,), (0,)), ((), ())),
                preferred_element_type=jnp.float32)
            s_dy = _shift_lanes(s_dy, _X * dy)
            acc = s_dy if acc is None else acc + s_dy

        if dilate:
            mdz = jnp.maximum(
                jnp.maximum(ring_m[pl.ds(s_m, 1)][0, 1:2] * vzm,
                            ring_m[pl.ds(s_0, 1)][0, 1:2]),
                ring_m[pl.ds(s_p, 1)][0, 1:2] * vzp)
            m_out = jnp.maximum(
                jnp.maximum(_shift_lanes(mdz, -_X), mdz),
                _shift_lanes(mdz, _X))
            maskout_ref[0] = m_out
        else:
            m_out = ring_m[pl.ds(s_0, 1)][0, 0:1]

        if kind == "last":
            out_ref[0] = acc.T
        else:
            out_ref[0] = acc

        n = jnp.sum(m_out)
        s1 = jnp.sum(acc * m_out, axis=1)
        s2 = jnp.sum(acc * acc * m_out, axis=1)
        stats_ref[...] += jnp.concatenate(
            [jnp.broadcast_to(n, (1, _CMID)),
             s1.reshape(1, _CMID), s2.reshape(1, _CMID)], axis=0)

        if kind == "last":
            @pl.when(i == _BZ)
            def _():
                st6 = stats_ref[...]
                mu6 = st6[1:2] / st6[0:1]
                var6 = st6[2:3] / st6[0:1] - mu6 * mu6
                sc6 = gn_ref[...] * lax.rsqrt(var6 + _EPS)
                scsh_ref[...] = jnp.concatenate(
                    [sc6, bn_ref[...] - mu6 * sc6], axis=0)


def _conv_layer(x, mask, scale, shift, wmat, kind, cin):
    def zin(i):
        return (jnp.clip(i, 0, _BZ - 1), 0, 0)

    def zin34(i):
        t = jnp.clip(i, 0, _BZ - 1)
        return (t + (t >= 16).astype(jnp.int32), 0, 0)

    def zout(i):
        return (jnp.clip(i - 1, 0, _BZ - 1), 0, 0)

    if kind == "first":
        xspec = pl.BlockSpec((1, _YX, 16), zin34)
        mspec = pl.BlockSpec((1, _YX, 16), zin34)
    else:
        xspec = pl.BlockSpec((1, _CMID, _YX), zin)
        mspec = pl.BlockSpec((1, 1, _YX), zin)
    full2 = lambda shape: pl.BlockSpec(shape, lambda i: (0, 0))

    in_specs = [xspec, mspec]
    inputs = [x, mask]
    if kind != "first":
        st_prev, g_prev, b_prev = scale
        in_specs += [full2((3, cin)), full2((1, cin)), full2((1, cin))]
        inputs += [st_prev, g_prev.reshape(1, cin), b_prev.reshape(1, cin)]
    if kind == "last":
        gn, bn = shift
        in_specs += [full2((1, _CMID)), full2((1, _CMID))]
        inputs += [gn.reshape(1, _CMID), bn.reshape(1, _CMID)]
    in_specs.append(full2(wmat.shape))
    inputs.append(wmat)
    mrows = 2 if kind in ("first", "dilate") else 1
    scratch_shapes = [
        pltpu.VMEM((3, 3 * cin, _YX), jnp.float32),
        pltpu.VMEM((3, mrows, _YX), jnp.float32),
    ]

    if kind == "last":
        out_shape = [jax.ShapeDtypeStruct((_BZ, _YX, _CMID), jnp.float32)]
        out_specs = [pl.BlockSpec((1, _YX, _CMID), zout)]
    else:
        out_shape = [jax.ShapeDtypeStruct((_BZ, _CMID, _YX), jnp.float32)]
        out_specs = [pl.BlockSpec((1, _CMID, _YX), zout)]
    if kind in ("first", "dilate"):
        out_shape.append(jax.ShapeDtypeStruct((_BZ, 1, _YX), jnp.float32))
        out_specs.append(pl.BlockSpec((1, 1, _YX), zout))
    out_shape.append(jax.ShapeDtypeStruct((3, _CMID), jnp.float32))
    out_specs.append(pl.BlockSpec((3, _CMID), lambda i: (0, 0)))
    if kind == "last":
        out_shape.append(jax.ShapeDtypeStruct((2, _CMID), jnp.float32))
        out_specs.append(pl.BlockSpec((2, _CMID), lambda i: (0, 0)))

    outs = pl.pallas_call(
        functools.partial(_conv_body, kind, cin),
        grid=(_BZ + 1,),
        in_specs=in_specs,
        out_specs=out_specs,
        out_shape=out_shape,
        scratch_shapes=scratch_shapes,
    )(*inputs)

    if kind in ("first", "dilate"):
        out, mask_out, stats = outs
        return out, mask_out, stats
    if kind == "last":
        out, stats, scsh = outs
        return out, scsh, stats
    out, stats = outs
    return out, mask, stats


def _flatten_coords(c):
    return ((c[:, 0] * _Z + c[:, 1]) * _Y + c[:, 2]) * _X + c[:, 3]


def kernel(voxel_features, voxel_indices, coords1, coords2, W1a, g1a, b1a,
           W1b, g1b, b1b, W1c, g1c, b1c, W2a, g2a, b2a, W2b, g2b, b2b,
           W2c, g2c, b2c):
    n0 = voxel_features.shape[0]
    n2 = coords2.shape[0]

    flat0 = _flatten_coords(voxel_indices)
    row = flat0 + _YX * (flat0 >= 16 * _YX).astype(jnp.int32)
    rowp = jnp.concatenate(
        [row, jnp.full((_P0 - n0,), -1, jnp.int32)])
    jm = jnp.arange(_P0, dtype=jnp.int32) % _YX
    idx2c = jnp.stack([
        jnp.where((rowp >= 0) & (rowp < _HALF), rowp, 16 * _YX + jm),
        jnp.where(rowp >= _HALF, rowp, 33 * _YX + jm),
    ]).reshape(2, _P0 // _CHUNK, _CHUNK)
    vf_pad = jnp.concatenate(
        [voxel_features, jnp.zeros((_P0 - n0, 16), jnp.float32)])

    feat_d, mask_d = _sc_scatter_dense(vf_pad, idx2c)
    x0 = feat_d.reshape(_NSL, _YX, 16)
    m0 = mask_d.reshape(_NSL, _YX, 16)

    def wm(W, cin):
        return W.reshape(3, 3, 3, cin, _CMID).transpose(
            1, 0, 2, 3, 4).reshape(27 * cin, _CMID).T

    o1, m1, s1 = _conv_layer(x0, m0, None, None, wm(W1a, 16), "first", 16)
    o2, _, s2 = _conv_layer(o1, m1, (s1, g1a, b1a), None,
                            wm(W1b, _CMID), "mid", _CMID)
    o3, _, s3 = _conv_layer(o2, m1, (s2, g1b, b1b), None,
                            wm(W1c, _CMID), "mid", _CMID)
    o4, m2, s4 = _conv_layer(o3, m1, (s3, g1c, b1c), None,
                             wm(W2a, _CMID), "dilate", _CMID)
    o5, _, s5 = _conv_layer(o4, m2, (s4, g2a, b2a), None,
                            wm(W2b, _CMID), "mid", _CMID)
    o6, scsh6, s6 = _conv_layer(o5, m2, (s5, g2b, b2b), (g2c, b2c),
                                wm(W2c, _CMID), "last", _CMID)

    flat2 = _flatten_coords(coords2)
    p2 = ((n2 + _NW * _SUP - 1) // (_NW * _SUP)) * (_NW * _SUP)
    idx2 = jnp.concatenate(
        [flat2, jnp.zeros((p2 - n2,), jnp.int32)]) if p2 != n2 else flat2
    rows6 = o6.reshape(_NSITE, _CMID)
    out = _sc_gather_rows(rows6, idx2.reshape(p2 // _CHUNK, _CHUNK),
                          scsh6[0], scsh6[1])
    return out[:n2]
